# Initial kernel scaffold; baseline (speedup 1.0000x reference)
import functools

import jax
import jax.numpy as jnp
from jax import lax
from jax.experimental import pallas as pl
from jax.experimental.pallas import tpu as pltpu
from jax.experimental.pallas import tpu_sc as plsc

B = 16384
K = 50
NUM_TYPES = 27
GLOVE_DIM = 300
EMBED_DIM = 16

NW = 32
CHUNK = 128
N_TOTAL = B * K
PER_W = N_TOTAL // NW
N_CHUNKS = PER_W // CHUNK


def _table_body(glove_ref, w_ref, b_ref, table_ref):
    g = glove_ref[...]
    w = w_ref[...]
    t = lax.dot_general(g, w, (((1,), (1,)), ((), ())),
                        preferred_element_type=jnp.float32)
    table_ref[...] = t + b_ref[...]


def _project_table(glove, W, b):
    return pl.pallas_call(
        _table_body,
        out_shape=jax.ShapeDtypeStruct((NUM_TYPES, EMBED_DIM), jnp.float32),
    )(glove, W, b.reshape(1, EMBED_DIM))


def _sc_gather_body(table_hbm, idx_hbm, out_hbm, idx_v, rows_v, sem):
    wid = lax.axis_index("s") * 2 + lax.axis_index("c")
    pltpu.sync_copy(idx_hbm.at[wid], idx_v)

    def chunk(j, carry):
        pltpu.async_copy(table_hbm.at[idx_v.at[j]], rows_v, sem).wait()
        pltpu.sync_copy(rows_v, out_hbm.at[wid, j])
        return carry

    lax.fori_loop(0, N_CHUNKS, chunk, 0)


_sc_gather = functools.partial(
    pl.kernel,
    out_type=jax.ShapeDtypeStruct((NW, N_CHUNKS, CHUNK, EMBED_DIM),
                                  jnp.float32),
    mesh=plsc.VectorSubcoreMesh(core_axis_name="c", subcore_axis_name="s"),
    scratch_types=[
        pltpu.VMEM((N_CHUNKS, CHUNK), jnp.int32),
        pltpu.VMEM((CHUNK, EMBED_DIM), jnp.float32),
        pltpu.SemaphoreType.DMA,
    ],
)(_sc_gather_body)


@jax.jit
def kernel(type_indices, glove_embeddings, W, b):
    table = _project_table(glove_embeddings, W, b)
    idx = type_indices.reshape(NW, N_CHUNKS, CHUNK).astype(jnp.int32)
    out = _sc_gather(table, idx)
    return out.reshape(B, K, EMBED_DIM)


# speedup vs baseline: 2.6716x; 2.6716x over previous
import functools

import jax
import jax.numpy as jnp
from jax import lax
from jax.experimental import pallas as pl
from jax.experimental.pallas import tpu as pltpu
from jax.experimental.pallas import tpu_sc as plsc

B = 16384
K = 50
NUM_TYPES = 27
GLOVE_DIM = 300
EMBED_DIM = 16

NW = 32
CHUNK = 128
N_TOTAL = B * K
PER_W = N_TOTAL // NW
N_CHUNKS = PER_W // CHUNK


def _table_body(glove_ref, w_ref, b_ref, table_ref):
    g = glove_ref[...]
    w = w_ref[...]
    t = lax.dot_general(g, w, (((1,), (1,)), ((), ())),
                        preferred_element_type=jnp.float32)
    table_ref[...] = t + b_ref[...]


def _project_table(glove, W, b):
    return pl.pallas_call(
        _table_body,
        out_shape=jax.ShapeDtypeStruct((NUM_TYPES, EMBED_DIM), jnp.float32),
    )(glove, W, b.reshape(1, EMBED_DIM))


def _sc_gather_body(table_hbm, idx_hbm, out_hbm, idx_v, rows_v, sem):
    wid = lax.axis_index("s") * 2 + lax.axis_index("c")
    pltpu.sync_copy(idx_hbm.at[wid], idx_v)

    def chunk(j, carry):
        pltpu.async_copy(table_hbm.at[idx_v.at[j]], rows_v, sem).wait()
        pltpu.sync_copy(rows_v, out_hbm.at[wid, j])
        return carry

    lax.fori_loop(0, N_CHUNKS, chunk, 0)


_sc_gather = functools.partial(
    pl.kernel,
    out_type=jax.ShapeDtypeStruct((NW, N_CHUNKS, CHUNK, EMBED_DIM),
                                  jnp.float32),
    mesh=plsc.VectorSubcoreMesh(core_axis_name="c", subcore_axis_name="s"),
    scratch_types=[
        pltpu.VMEM((N_CHUNKS, CHUNK), jnp.int32),
        pltpu.VMEM((CHUNK, EMBED_DIM), jnp.float32),
        pltpu.SemaphoreType.DMA,
    ],
    compiler_params=pltpu.CompilerParams(use_tc_tiling_on_sc=False),
)(_sc_gather_body)


@jax.jit
def kernel(type_indices, glove_embeddings, W, b):
    table = _project_table(glove_embeddings, W, b)
    idx = type_indices.reshape(NW, N_CHUNKS, CHUNK).astype(jnp.int32)
    out = _sc_gather(table, idx)
    return out.reshape(B, K, EMBED_DIM)
